# Initial kernel scaffold; baseline (speedup 1.0000x reference)
import jax
import jax.numpy as jnp
from jax.experimental import pallas as pl


def kernel(gt_depths, images, inputs_normal, targets_normal):
    raise NotImplementedError("write your pallas kernel here")


# speedup vs baseline: 2.6162x; 2.6162x over previous
import functools

import jax
import jax.numpy as jnp
from jax import lax
from jax.experimental import pallas as pl
from jax.experimental.pallas import tpu as pltpu
from jax.experimental.pallas import tpu_sc as plsc

H = W = 384
N = H * W
NW = 32
PER_W = N // NW
CH = 1536
NCHUNK = PER_W // CH


def _conv(x, kern, padding=0, groups=1):
    return lax.conv_general_dilated(
        x, kern, window_strides=(1, 1),
        padding=((padding, padding), (padding, padding)),
        dimension_numbers=('NCHW', 'OIHW', 'NCHW'),
        feature_group_count=groups)


def _sob():
    a = jnp.array([[-1.0, 0.0, 1.0], [-2.0, 0.0, 2.0], [-1.0, 0.0, 1.0]],
                  jnp.float32).reshape(1, 1, 3, 3)
    b = jnp.array([[1.0, 2.0, 1.0], [0.0, 0.0, 0.0], [-1.0, -2.0, -1.0]],
                  jnp.float32).reshape(1, 1, 3, 3)
    return a, b


def _edge(images):
    n, c, h, w = images.shape
    a, b = _sob()
    src = images[:, 0:1, :, :] if c == 3 else images
    gx = _conv(src, a)
    gy = _conv(src, b)
    edges = jnp.pad(jnp.sqrt(gx * gx + gy * gy), ((0, 0), (0, 0), (1, 1), (1, 1)))
    thetas = jnp.pad(jnp.arctan2(gy, gx), ((0, 0), (0, 0), (1, 1), (1, 1)))
    return edges, thetas


def _normal_edge(normals):
    n, c, h, w = normals.shape
    a, b = _sob()
    a3 = jnp.tile(a, (c, 1, 1, 1))
    b3 = jnp.tile(b, (c, 1, 1, 1))
    gx = jnp.abs(_conv(normals, a3, groups=c)).mean(axis=1, keepdims=True)
    gy = jnp.abs(_conv(normals, b3, groups=c)).mean(axis=1, keepdims=True)
    edges = jnp.pad(jnp.sqrt(gx * gx + gy * gy), ((0, 0), (0, 0), (1, 1), (1, 1)))
    thetas = jnp.pad(jnp.arctan2(gy, gx), ((0, 0), (0, 0), (1, 1), (1, 1)))
    return edges, thetas


def _sample(edges_flat, thetas_flat, h, w, key):
    max_num = edges_flat.shape[0]
    edges_max = edges_flat.max()
    edges_mask = edges_flat >= edges_max * 0.1
    count = jnp.sum(edges_mask)
    edges_loc = jnp.nonzero(edges_mask, size=max_num, fill_value=0)[0]
    thetas_edge = thetas_flat[edges_loc]
    k1, k2 = jax.random.split(key)
    index_anchors = jax.random.randint(k1, (max_num,), 0, count)
    theta_anchors = thetas_edge[index_anchors]
    flat = edges_loc[index_anchors]
    row_anchors = flat // w
    col_anchors = flat - row_anchors * w
    distance_flat = jax.random.randint(k2, (4 * max_num,), 3, 20)
    gather_idx = jnp.arange(4)[:, None] * count + jnp.arange(max_num)[None, :]
    distance = distance_flat[gather_idx].astype(jnp.float32)
    pos_or_neg = jnp.ones((4, max_num), jnp.float32)
    pos_or_neg = pos_or_neg.at[:2, :].set(-1.0)
    distance = distance * pos_or_neg
    col = col_anchors[None, :] + jnp.round(
        distance * jnp.abs(jnp.cos(theta_anchors))[None, :]).astype(flat.dtype)
    row = row_anchors[None, :] + jnp.round(
        distance * jnp.abs(jnp.sin(theta_anchors))[None, :]).astype(flat.dtype)
    col = jnp.clip(col, 0, w - 1)
    row = jnp.clip(row, 0, h - 1)
    return row * w + col, count


def _sc_pair_loss(table, idx_full):
    mesh = plsc.VectorSubcoreMesh(core_axis_name="c", subcore_axis_name="s")

    @functools.partial(
        pl.kernel, mesh=mesh,
        compiler_params=pltpu.CompilerParams(needs_layout_passes=False,
                                             use_tc_tiling_on_sc=False),
        out_type=jax.ShapeDtypeStruct((NW, 16), jnp.float32),
        scratch_types=[
            pltpu.VMEM((4, CH), jnp.int32),
            pltpu.VMEM((4, CH, 8), jnp.float32),
            pltpu.VMEM((16,), jnp.float32),
            pltpu.SemaphoreType.DMA,
        ],
    )
    def k(table_hbm, idx_hbm, out_hbm, idx_v, rows_v, acc_v, sem):
        wid = lax.axis_index("s") * 2 + lax.axis_index("c")
        zero = jnp.zeros((16,), jnp.float32)

        def chunk_body(t, acc):
            g = t // NCHUNK
            ci = t - g * NCHUNK
            base = wid * PER_W + ci * CH
            for j in range(4):
                pltpu.sync_copy(idx_hbm.at[g, j, pl.ds(base, CH)], idx_v.at[j])
            cps = [pltpu.async_copy(table_hbm.at[idx_v.at[j]], rows_v.at[j], sem)
                   for j in range(4)]
            for cp in cps:
                cp.wait()

            def step(p, a):
                rid = lax.iota(jnp.int32, 16) + p * 16
                v = [[plsc.load_gather(rows_v.at[j],
                                       [rid, jnp.full((16,), c, jnp.int32)])
                      for c in range(6)] for j in range(4)]
                for (x, y) in ((0, 1), (1, 2), (2, 3)):
                    tcos = jnp.abs(v[x][0] * v[y][0] + v[x][1] * v[y][1]
                                   + v[x][2] * v[y][2])
                    icos = jnp.abs(v[x][3] * v[y][3] + v[x][4] * v[y][4]
                                   + v[x][5] * v[y][5])
                    a = a + jnp.abs(tcos - icos)
                return a

            return lax.fori_loop(0, CH // 16, step, acc)

        acc = lax.fori_loop(0, 4 * NCHUNK, chunk_body, zero)
        acc_v[...] = acc
        pltpu.sync_copy(acc_v, out_hbm.at[wid])

    return k(table, idx_full)


def kernel(gt_depths, images, inputs_normal, targets_normal):
    n, c, h, w = targets_normal.shape

    edges_img, thetas_img = _edge(images)
    edges_normal, thetas_normal = _normal_edge(targets_normal)
    border = jnp.ones_like(edges_normal)
    border = border.at[:, :, 5:-5, 5:-5].set(0.0)
    edges_normal = jnp.where(border.astype(bool), 0.0, edges_normal)
    edges_depth, _ = _edge(gt_depths)
    edges_depth_mask = edges_depth >= edges_depth.max() * 0.1
    dil_k = jnp.ones((1, 1, 3, 3), jnp.float32)
    dilate = jnp.clip(_conv(edges_depth_mask.astype(jnp.float32), dil_k,
                            padding=1), 0.0, 1.0).astype(bool)
    edges_normal = jnp.where(dilate, 0.0, edges_normal)
    edges_img = jnp.where(dilate, 0.0, edges_img)

    edges_img_f = edges_img.reshape(n, -1)
    thetas_img_f = thetas_img.reshape(n, -1)
    edges_normal_f = edges_normal.reshape(n, -1)
    thetas_normal_f = thetas_normal.reshape(n, -1)

    key = jax.random.key(42)
    idx_groups = []
    counts = []
    p_arange = jnp.arange(N, dtype=jnp.int32)
    for i in range(n):
        for s in range(2):
            if s == 0:
                idx, cnt = _sample(edges_img_f[i], thetas_img_f[i], h, w,
                                   jax.random.fold_in(key, 2 * i))
            else:
                idx, cnt = _sample(edges_normal_f[i], thetas_normal_f[i], h, w,
                                   jax.random.fold_in(key, 2 * i + 1))
            idx_off = jnp.where(p_arange[None, :] < cnt,
                                idx.astype(jnp.int32) + i * N,
                                jnp.int32(2 * N))
            idx_groups.append(idx_off)
            counts.append(cnt)

    idx_full = jnp.stack(idx_groups, axis=0)
    t_rows = targets_normal.reshape(n, c, N).transpose(0, 2, 1)
    i_rows = inputs_normal.reshape(n, c, N).transpose(0, 2, 1)
    packed = jnp.concatenate(
        [t_rows, i_rows, jnp.zeros((n, N, 2), jnp.float32)], axis=-1)
    table = jnp.concatenate(
        [packed.reshape(n * N, 8), jnp.zeros((16, 8), jnp.float32)], axis=0)

    partial = _sc_pair_loss(table, idx_full)
    total_count = 3.0 * jnp.sum(jnp.stack(counts)).astype(jnp.float32)
    return jnp.sum(partial) / total_count


# speedup vs baseline: 2.9282x; 1.1193x over previous
import functools

import jax
import jax.numpy as jnp
from jax import lax
from jax.experimental import pallas as pl
from jax.experimental.pallas import tpu as pltpu
from jax.experimental.pallas import tpu_sc as plsc

H = W = 384
N = H * W
NW = 32
PER_W = N // NW
CH = 1536
NCHUNK = PER_W // CH


def _conv(x, kern, padding=0, groups=1):
    return lax.conv_general_dilated(
        x, kern, window_strides=(1, 1),
        padding=((padding, padding), (padding, padding)),
        dimension_numbers=('NCHW', 'OIHW', 'NCHW'),
        feature_group_count=groups)


def _sob():
    a = jnp.array([[-1.0, 0.0, 1.0], [-2.0, 0.0, 2.0], [-1.0, 0.0, 1.0]],
                  jnp.float32).reshape(1, 1, 3, 3)
    b = jnp.array([[1.0, 2.0, 1.0], [0.0, 0.0, 0.0], [-1.0, -2.0, -1.0]],
                  jnp.float32).reshape(1, 1, 3, 3)
    return a, b


def _edge(images):
    n, c, h, w = images.shape
    a, b = _sob()
    src = images[:, 0:1, :, :] if c == 3 else images
    gx = _conv(src, a)
    gy = _conv(src, b)
    edges = jnp.pad(jnp.sqrt(gx * gx + gy * gy), ((0, 0), (0, 0), (1, 1), (1, 1)))
    thetas = jnp.pad(jnp.arctan2(gy, gx), ((0, 0), (0, 0), (1, 1), (1, 1)))
    return edges, thetas


def _normal_edge(normals):
    n, c, h, w = normals.shape
    a, b = _sob()
    a3 = jnp.tile(a, (c, 1, 1, 1))
    b3 = jnp.tile(b, (c, 1, 1, 1))
    gx = jnp.abs(_conv(normals, a3, groups=c)).mean(axis=1, keepdims=True)
    gy = jnp.abs(_conv(normals, b3, groups=c)).mean(axis=1, keepdims=True)
    edges = jnp.pad(jnp.sqrt(gx * gx + gy * gy), ((0, 0), (0, 0), (1, 1), (1, 1)))
    thetas = jnp.pad(jnp.arctan2(gy, gx), ((0, 0), (0, 0), (1, 1), (1, 1)))
    return edges, thetas


def _sample(edges_flat, thetas_flat, h, w, key):
    max_num = edges_flat.shape[0]
    edges_max = edges_flat.max()
    edges_mask = edges_flat >= edges_max * 0.1
    count = jnp.sum(edges_mask)
    edges_loc = jnp.nonzero(edges_mask, size=max_num, fill_value=0)[0]
    thetas_edge = thetas_flat[edges_loc]
    k1, k2 = jax.random.split(key)
    index_anchors = jax.random.randint(k1, (max_num,), 0, count)
    theta_anchors = thetas_edge[index_anchors]
    flat = edges_loc[index_anchors]
    row_anchors = flat // w
    col_anchors = flat - row_anchors * w
    distance_flat = jax.random.randint(k2, (4 * max_num,), 3, 20)
    gather_idx = jnp.arange(4)[:, None] * count + jnp.arange(max_num)[None, :]
    distance = distance_flat[gather_idx].astype(jnp.float32)
    pos_or_neg = jnp.ones((4, max_num), jnp.float32)
    pos_or_neg = pos_or_neg.at[:2, :].set(-1.0)
    distance = distance * pos_or_neg
    col = col_anchors[None, :] + jnp.round(
        distance * jnp.abs(jnp.cos(theta_anchors))[None, :]).astype(flat.dtype)
    row = row_anchors[None, :] + jnp.round(
        distance * jnp.abs(jnp.sin(theta_anchors))[None, :]).astype(flat.dtype)
    col = jnp.clip(col, 0, w - 1)
    row = jnp.clip(row, 0, h - 1)
    return row * w + col, count


CB = 768
NCH = N // 16 // CB
SUB = 384


def _sc_pair_loss(table, idx_full):
    mesh = plsc.VectorSubcoreMesh(core_axis_name="c", subcore_axis_name="s")

    @functools.partial(
        pl.kernel, mesh=mesh,
        compiler_params=pltpu.CompilerParams(needs_layout_passes=False,
                                             use_tc_tiling_on_sc=False),
        out_type=jax.ShapeDtypeStruct((NW, 16), jnp.float32),
        scratch_types=[
            pltpu.VMEM((2, 4, CB), jnp.int32),
            pltpu.VMEM((2, 4, CB, 8), jnp.float32),
            pltpu.VMEM((16,), jnp.float32),
            pltpu.VMEM_SHARED((N + 16, 8), jnp.float32),
            pltpu.SemaphoreType.DMA,
            pltpu.SemaphoreType.DMA,
            pltpu.SemaphoreType.DMA,
        ],
    )
    def k(table_hbm, idx_hbm, out_hbm, idx_v, rows_v, acc_v, sh_table,
          semg0, semg1, sems):
        cid = lax.axis_index("c")
        sid = lax.axis_index("s")
        wid = sid * 2 + cid

        @pl.when(sid == 0)
        def _():
            pltpu.sync_copy(table_hbm.at[pl.ds(cid * N, N)],
                            sh_table.at[pl.ds(0, N)])
            pltpu.sync_copy(table_hbm.at[pl.ds(2 * N, 16)],
                            sh_table.at[pl.ds(N, 16)])
        plsc.subcore_barrier()

        chunks = [(gl, ci) for gl in range(2) for ci in range(NCH)]

        def stage_idx(t, b):
            gl, ci = chunks[t]
            base = sid * (NCH * CB) + ci * CB
            return [pltpu.async_copy(
                idx_hbm.at[2 * cid + gl, j, pl.ds(base, CB)],
                idx_v.at[b, j], sems) for j in range(4)]

        def fire_gathers(b):
            semg = semg0 if b == 0 else semg1
            return [pltpu.async_copy(
                sh_table.at[idx_v.at[b, j, pl.ds(s * SUB, SUB)]],
                rows_v.at[b, j, pl.ds(s * SUB, SUB), :], semg)
                for j in range(4) for s in range(CB // SUB)]

        def compute(b, acc):
            def step(p, a):
                rid = lax.iota(jnp.int32, 16) + p * 16
                v = [[plsc.load_gather(rows_v.at[b, j],
                                       [rid, jnp.full((16,), c, jnp.int32)])
                      for c in range(6)] for j in range(4)]
                for (x, y) in ((0, 1), (1, 2), (2, 3)):
                    tcos = jnp.abs(v[x][0] * v[y][0] + v[x][1] * v[y][1]
                                   + v[x][2] * v[y][2])
                    icos = jnp.abs(v[x][3] * v[y][3] + v[x][4] * v[y][4]
                                   + v[x][5] * v[y][5])
                    a = a + jnp.abs(tcos - icos)
                return a
            return lax.fori_loop(0, CB // 16, step, acc)

        acc = jnp.zeros((16,), jnp.float32)
        ntot = len(chunks)
        for cp in stage_idx(0, 0):
            cp.wait()
        gcps = fire_gathers(0)
        for t in range(ntot):
            b = t & 1
            next_gcps = None
            if t + 1 < ntot:
                nb = (t + 1) & 1
                for cp in stage_idx(t + 1, nb):
                    cp.wait()
                next_gcps = fire_gathers(nb)
            for cp in gcps:
                cp.wait()
            acc = compute(b, acc)
            gcps = next_gcps
        acc_v[...] = acc
        pltpu.sync_copy(acc_v, out_hbm.at[wid])

    return k(table, idx_full)


def kernel(gt_depths, images, inputs_normal, targets_normal):
    n, c, h, w = targets_normal.shape

    edges_img, thetas_img = _edge(images)
    edges_normal, thetas_normal = _normal_edge(targets_normal)
    border = jnp.ones_like(edges_normal)
    border = border.at[:, :, 5:-5, 5:-5].set(0.0)
    edges_normal = jnp.where(border.astype(bool), 0.0, edges_normal)
    edges_depth, _ = _edge(gt_depths)
    edges_depth_mask = edges_depth >= edges_depth.max() * 0.1
    dil_k = jnp.ones((1, 1, 3, 3), jnp.float32)
    dilate = jnp.clip(_conv(edges_depth_mask.astype(jnp.float32), dil_k,
                            padding=1), 0.0, 1.0).astype(bool)
    edges_normal = jnp.where(dilate, 0.0, edges_normal)
    edges_img = jnp.where(dilate, 0.0, edges_img)

    edges_img_f = edges_img.reshape(n, -1)
    thetas_img_f = thetas_img.reshape(n, -1)
    edges_normal_f = edges_normal.reshape(n, -1)
    thetas_normal_f = thetas_normal.reshape(n, -1)

    key = jax.random.key(42)
    idx_groups = []
    counts = []
    p_arange = jnp.arange(N, dtype=jnp.int32)
    for i in range(n):
        for s in range(2):
            if s == 0:
                idx, cnt = _sample(edges_img_f[i], thetas_img_f[i], h, w,
                                   jax.random.fold_in(key, 2 * i))
            else:
                idx, cnt = _sample(edges_normal_f[i], thetas_normal_f[i], h, w,
                                   jax.random.fold_in(key, 2 * i + 1))
            idx_off = jnp.where(p_arange[None, :] < cnt,
                                idx.astype(jnp.int32), jnp.int32(N))
            idx_groups.append(idx_off)
            counts.append(cnt)

    idx_full = jnp.stack(idx_groups, axis=0)
    t_rows = targets_normal.reshape(n, c, N).transpose(0, 2, 1)
    i_rows = inputs_normal.reshape(n, c, N).transpose(0, 2, 1)
    packed = jnp.concatenate(
        [t_rows, i_rows, jnp.zeros((n, N, 2), jnp.float32)], axis=-1)
    table = jnp.concatenate(
        [packed.reshape(n * N, 8), jnp.zeros((16, 8), jnp.float32)], axis=0)

    partial = _sc_pair_loss(table, idx_full)
    total_count = 3.0 * jnp.sum(jnp.stack(counts)).astype(jnp.float32)
    return jnp.sum(partial) / total_count
